# baseline (speedup 1.0000x reference)
import functools

import jax
import jax.numpy as jnp
from jax import lax
from jax.experimental import pallas as pl
from jax.experimental.pallas import tpu as pltpu
from jax.experimental.pallas import tpu_sc as plsc

N_ROWS = 50000
N_COLS = 80
CHUNK_ROWS = 80
N_CHUNKS = N_ROWS // CHUNK_ROWS
N_WORKERS = 32
MAX_CHUNKS_PER_WORKER = 20

_L1P = (2.215976490638205e-07, 0.9999702432977314, -0.4993339489819427,
        0.32751171370201704, -0.22396689943036466, 0.13198966240066795,
        -0.05326747773448861, 0.01024382863145101)
_RCP = (0.9999987181130621, -0.9998266719607205, 0.9960836138976835,
        -0.9653065806642992, 0.8417206934747233, -0.5733132652071123,
        0.2513469507485837, -0.050704364832866916)


def _poly(u, coefs):
    p = jnp.full((16,), coefs[-1], jnp.float32)
    for c in coefs[-2::-1]:
        p = p * u + jnp.float32(c)
    return p


def _base_sig(x):
    u = jnp.exp(-jnp.abs(x))
    base = jnp.maximum(x, jnp.float32(0)) + _poly(u, _L1P)
    r = _poly(u, _RCP)
    sig = jnp.where(x >= jnp.float32(0), r, u * r)
    return base, sig


def _qfl_body(x_hbm, lbl_hbm, sco_hbm, wgt_hbm, out_hbm,
              xb0, xb1, lb0, lb1, sb0, sb1, wb0, wb1, acc_ref,
              sem0, sem1):
    core = lax.axis_index("c")
    sub = lax.axis_index("s")
    wid = sub * 2 + core
    nch = jnp.where(wid < 17, 20, 19)

    iota = lax.iota(jnp.int32, 16)
    acc_ref[...] = jnp.zeros((16,), jnp.float32)

    bufs = ((xb0, lb0, sb0, wb0, sem0), (xb1, lb1, sb1, wb1, sem1))

    def issue(n, slot):
        xb, lb, sb, wb, sem = bufs[slot]
        cid = wid + n * N_WORKERS
        r0 = cid * CHUNK_ROWS
        pltpu.async_copy(x_hbm.at[pl.ds(r0 * N_COLS, CHUNK_ROWS * N_COLS)], xb, sem)
        pltpu.async_copy(lbl_hbm.at[pl.ds(r0, CHUNK_ROWS)], lb, sem)
        pltpu.async_copy(sco_hbm.at[pl.ds(r0, CHUNK_ROWS)], sb, sem)
        pltpu.async_copy(wgt_hbm.at[pl.ds(r0, CHUNK_ROWS)], wb, sem)

    def wait(slot):
        xb, lb, sb, wb, sem = bufs[slot]
        pltpu.make_async_copy(x_hbm.at[pl.ds(0, CHUNK_ROWS * N_COLS)], xb, sem).wait()
        pltpu.make_async_copy(lbl_hbm.at[pl.ds(0, CHUNK_ROWS)], lb, sem).wait()
        pltpu.make_async_copy(sco_hbm.at[pl.ds(0, CHUNK_ROWS)], sb, sem).wait()
        pltpu.make_async_copy(wgt_hbm.at[pl.ds(0, CHUNK_ROWS)], wb, sem).wait()

    def process(slot):
        xb, lb, sb, wb, _ = bufs[slot]
        for g in range(CHUNK_ROWS // 16):
            fbase = (g * 16 + iota) * N_COLS

            def blk_body(b, accs):
                accs = list(accs)
                fb = fbase + b * 16
                for k in range(16):
                    x = plsc.load_gather(xb, [fb + k])
                    base, sig = _base_sig(x)
                    accs[k % 4] = accs[k % 4] + base * sig * sig
                return tuple(accs)

            zeros = jnp.zeros((16,), jnp.float32)
            a0, a1, a2, a3 = lax.fori_loop(0, N_COLS // 16, blk_body,
                                           (zeros, zeros, zeros, zeros))
            acc = (a0 + a1) + (a2 + a3)
            lbl = lb[pl.ds(g * 16, 16)]
            sco = sb[pl.ds(g * 16, 16)]
            wgt = wb[pl.ds(g * 16, 16)]
            mask = (lbl >= 0) & (lbl < N_COLS)
            safe = jnp.where(mask, lbl, 0)
            xp = plsc.load_gather(xb, [fbase + safe])
            bp, sp = _base_sig(xp)
            d = sco - sp
            corr = (bp - xp * sco) * d * d - bp * sp * sp
            tot = acc + jnp.where(mask, corr, jnp.float32(0))
            acc_ref[...] += tot * wgt

    issue(0, 0)

    def pair_body(i, carry):
        @pl.when(2 * i + 1 < nch)
        def _():
            issue(2 * i + 1, 1)
        wait(0)
        process(0)

        @pl.when(2 * i + 2 < nch)
        def _():
            issue(2 * i + 2, 0)

        @pl.when(2 * i + 1 < nch)
        def _():
            wait(1)
            process(1)
        return carry

    lax.fori_loop(0, MAX_CHUNKS_PER_WORKER // 2, pair_body, 0)

    pltpu.sync_copy(acc_ref, out_hbm.at[wid])


@functools.partial(jax.jit, static_argnames=())
def _qfl_partials(x, lbl, sco, wgt):
    kfn = pl.kernel(
        _qfl_body,
        out_type=jax.ShapeDtypeStruct((N_WORKERS, 16), jnp.float32),
        mesh=plsc.VectorSubcoreMesh(core_axis_name="c", subcore_axis_name="s"),
        compiler_params=pltpu.CompilerParams(needs_layout_passes=False),
        scratch_types=[
            pltpu.VMEM((CHUNK_ROWS * N_COLS,), jnp.float32),
            pltpu.VMEM((CHUNK_ROWS * N_COLS,), jnp.float32),
            pltpu.VMEM((CHUNK_ROWS,), jnp.int32),
            pltpu.VMEM((CHUNK_ROWS,), jnp.int32),
            pltpu.VMEM((CHUNK_ROWS,), jnp.float32),
            pltpu.VMEM((CHUNK_ROWS,), jnp.float32),
            pltpu.VMEM((CHUNK_ROWS,), jnp.float32),
            pltpu.VMEM((CHUNK_ROWS,), jnp.float32),
            pltpu.VMEM((16,), jnp.float32),
            pltpu.SemaphoreType.DMA,
            pltpu.SemaphoreType.DMA,
        ],
    )
    return kfn(x, lbl, sco, wgt)


def kernel(output, label, score, weight, avg_factor):
    partials = _qfl_partials(output.reshape(-1), label.astype(jnp.int32), score,
                             weight)
    return partials.sum() / avg_factor


# speedup vs baseline: 1.1491x; 1.1491x over previous
import functools

import jax
import jax.numpy as jnp
from jax import lax
from jax.experimental import pallas as pl
from jax.experimental.pallas import tpu as pltpu
from jax.experimental.pallas import tpu_sc as plsc

N_ROWS = 50000
N_COLS = 80
CHUNK_ROWS = 80
N_CHUNKS = N_ROWS // CHUNK_ROWS
N_WORKERS = 32
MAX_CHUNKS_PER_WORKER = 20

_L1P = (2.215976490638205e-07, 0.9999702432977314, -0.4993339489819427,
        0.32751171370201704, -0.22396689943036466, 0.13198966240066795,
        -0.05326747773448861, 0.01024382863145101)
_RCP = (0.9999987181130621, -0.9998266719607205, 0.9960836138976835,
        -0.9653065806642992, 0.8417206934747233, -0.5733132652071123,
        0.2513469507485837, -0.050704364832866916)


def _poly(u, coefs):
    p = jnp.full((16,), coefs[-1], jnp.float32)
    for c in coefs[-2::-1]:
        p = p * u + jnp.float32(c)
    return p


def _base_sig(x):
    u = jnp.exp(-jnp.abs(x))
    base = jnp.maximum(x, jnp.float32(0)) + _poly(u, _L1P)
    r = jnp.float32(1) / (jnp.float32(1) + u)
    sig = jnp.where(x >= jnp.float32(0), r, u * r)
    return base, sig


def _qfl_body(x_hbm, lbl_hbm, sco_hbm, wgt_hbm, out_hbm,
              xb0, xb1, lb0, lb1, sb0, sb1, wb0, wb1, acc_ref,
              sem0, sem1):
    core = lax.axis_index("c")
    sub = lax.axis_index("s")
    wid = sub * 2 + core
    nch = jnp.where(wid < 17, 20, 19)

    iota = lax.iota(jnp.int32, 16)
    acc_ref[...] = jnp.zeros((16,), jnp.float32)

    bufs = ((xb0, lb0, sb0, wb0, sem0), (xb1, lb1, sb1, wb1, sem1))

    def issue(n, slot):
        xb, lb, sb, wb, sem = bufs[slot]
        cid = wid + n * N_WORKERS
        r0 = cid * CHUNK_ROWS
        pltpu.async_copy(x_hbm.at[pl.ds(r0 * N_COLS, CHUNK_ROWS * N_COLS)], xb, sem)
        pltpu.async_copy(lbl_hbm.at[pl.ds(r0, CHUNK_ROWS)], lb, sem)
        pltpu.async_copy(sco_hbm.at[pl.ds(r0, CHUNK_ROWS)], sb, sem)
        pltpu.async_copy(wgt_hbm.at[pl.ds(r0, CHUNK_ROWS)], wb, sem)

    def wait(slot):
        xb, lb, sb, wb, sem = bufs[slot]
        pltpu.make_async_copy(x_hbm.at[pl.ds(0, CHUNK_ROWS * N_COLS)], xb, sem).wait()
        pltpu.make_async_copy(lbl_hbm.at[pl.ds(0, CHUNK_ROWS)], lb, sem).wait()
        pltpu.make_async_copy(sco_hbm.at[pl.ds(0, CHUNK_ROWS)], sb, sem).wait()
        pltpu.make_async_copy(wgt_hbm.at[pl.ds(0, CHUNK_ROWS)], wb, sem).wait()

    def process(slot):
        xb, lb, sb, wb, _ = bufs[slot]
        for g in range(CHUNK_ROWS // 16):
            fbase = (g * 16 + iota) * N_COLS

            def blk_body(b, accs):
                accs = list(accs)
                fb = fbase + b * 16
                for k in range(16):
                    x = plsc.load_gather(xb, [fb + k])
                    base, sig = _base_sig(x)
                    accs[k % 4] = accs[k % 4] + base * sig * sig
                return tuple(accs)

            zeros = jnp.zeros((16,), jnp.float32)
            a0, a1, a2, a3 = lax.fori_loop(0, N_COLS // 16, blk_body,
                                           (zeros, zeros, zeros, zeros))
            acc = (a0 + a1) + (a2 + a3)
            lbl = lb[pl.ds(g * 16, 16)]
            sco = sb[pl.ds(g * 16, 16)]
            wgt = wb[pl.ds(g * 16, 16)]
            mask = (lbl >= 0) & (lbl < N_COLS)
            safe = jnp.where(mask, lbl, 0)
            xp = plsc.load_gather(xb, [fbase + safe])
            bp, sp = _base_sig(xp)
            d = sco - sp
            corr = (bp - xp * sco) * d * d - bp * sp * sp
            tot = acc + jnp.where(mask, corr, jnp.float32(0))
            acc_ref[...] += tot * wgt

    issue(0, 0)

    def pair_body(i, carry):
        @pl.when(2 * i + 1 < nch)
        def _():
            issue(2 * i + 1, 1)
        wait(0)
        process(0)

        @pl.when(2 * i + 2 < nch)
        def _():
            issue(2 * i + 2, 0)

        @pl.when(2 * i + 1 < nch)
        def _():
            wait(1)
            process(1)
        return carry

    lax.fori_loop(0, MAX_CHUNKS_PER_WORKER // 2, pair_body, 0)

    pltpu.sync_copy(acc_ref, out_hbm.at[wid])


@functools.partial(jax.jit, static_argnames=())
def _qfl_partials(x, lbl, sco, wgt):
    kfn = pl.kernel(
        _qfl_body,
        out_type=jax.ShapeDtypeStruct((N_WORKERS, 16), jnp.float32),
        mesh=plsc.VectorSubcoreMesh(core_axis_name="c", subcore_axis_name="s"),
        compiler_params=pltpu.CompilerParams(needs_layout_passes=False),
        scratch_types=[
            pltpu.VMEM((CHUNK_ROWS * N_COLS,), jnp.float32),
            pltpu.VMEM((CHUNK_ROWS * N_COLS,), jnp.float32),
            pltpu.VMEM((CHUNK_ROWS,), jnp.int32),
            pltpu.VMEM((CHUNK_ROWS,), jnp.int32),
            pltpu.VMEM((CHUNK_ROWS,), jnp.float32),
            pltpu.VMEM((CHUNK_ROWS,), jnp.float32),
            pltpu.VMEM((CHUNK_ROWS,), jnp.float32),
            pltpu.VMEM((CHUNK_ROWS,), jnp.float32),
            pltpu.VMEM((16,), jnp.float32),
            pltpu.SemaphoreType.DMA,
            pltpu.SemaphoreType.DMA,
        ],
    )
    return kfn(x, lbl, sco, wgt)


def kernel(output, label, score, weight, avg_factor):
    partials = _qfl_partials(output.reshape(-1), label.astype(jnp.int32), score,
                             weight)
    return partials.sum() / avg_factor


# speedup vs baseline: 1.5743x; 1.3700x over previous
import functools

import jax
import jax.numpy as jnp
from jax import lax
from jax.experimental import pallas as pl
from jax.experimental.pallas import tpu as pltpu
from jax.experimental.pallas import tpu_sc as plsc

N_ROWS = 50000
N_COLS = 80
CHUNK_ROWS = 80
N_CHUNKS = N_ROWS // CHUNK_ROWS
N_WORKERS = 32
MAX_CHUNKS_PER_WORKER = 20

_L1P = (2.215976490638205e-07, 0.9999702432977314, -0.4993339489819427,
        0.32751171370201704, -0.22396689943036466, 0.13198966240066795,
        -0.05326747773448861, 0.01024382863145101)
_RCP = (0.9999987181130621, -0.9998266719607205, 0.9960836138976835,
        -0.9653065806642992, 0.8417206934747233, -0.5733132652071123,
        0.2513469507485837, -0.050704364832866916)


def _poly(u, coefs):
    p = jnp.full((16,), coefs[-1], jnp.float32)
    for c in coefs[-2::-1]:
        p = p * u + jnp.float32(c)
    return p


def _base_sig(x):
    u = jnp.exp(-jnp.abs(x))
    base = jnp.maximum(x, jnp.float32(0)) + _poly(u, _L1P)
    r = jnp.float32(1) / (jnp.float32(1) + u)
    sig = jnp.where(x >= jnp.float32(0), r, u * r)
    return base, sig


def _qfl_body(x_hbm, lbl_hbm, sco_hbm, wgt_hbm, out_hbm,
              xb0, xb1, lb0, lb1, sb0, sb1, wb0, wb1, acc_ref,
              sem0, sem1):
    core = lax.axis_index("c")
    sub = lax.axis_index("s")
    wid = sub * 2 + core
    nch = jnp.where(wid < 17, 20, 19)

    iota = lax.iota(jnp.int32, 16)
    acc_ref[...] = jnp.zeros((16,), jnp.float32)

    bufs = ((xb0, lb0, sb0, wb0, sem0), (xb1, lb1, sb1, wb1, sem1))

    def issue(n, slot):
        xb, lb, sb, wb, sem = bufs[slot]
        cid = wid + n * N_WORKERS
        r0 = cid * CHUNK_ROWS
        pltpu.async_copy(x_hbm.at[pl.ds(r0, CHUNK_ROWS)], xb, sem)
        pltpu.async_copy(lbl_hbm.at[pl.ds(r0, CHUNK_ROWS)], lb, sem)
        pltpu.async_copy(sco_hbm.at[pl.ds(r0, CHUNK_ROWS)], sb, sem)
        pltpu.async_copy(wgt_hbm.at[pl.ds(r0, CHUNK_ROWS)], wb, sem)

    def wait(slot):
        xb, lb, sb, wb, sem = bufs[slot]
        pltpu.make_async_copy(x_hbm.at[pl.ds(0, CHUNK_ROWS)], xb, sem).wait()
        pltpu.make_async_copy(lbl_hbm.at[pl.ds(0, CHUNK_ROWS)], lb, sem).wait()
        pltpu.make_async_copy(sco_hbm.at[pl.ds(0, CHUNK_ROWS)], sb, sem).wait()
        pltpu.make_async_copy(wgt_hbm.at[pl.ds(0, CHUNK_ROWS)], wb, sem).wait()

    def process(slot):
        xb, lb, sb, wb, _ = bufs[slot]
        for g in range(CHUNK_ROWS // 16):
            rowv = g * 16 + iota

            def blk_body(b, accs):
                accs = list(accs)
                c0 = b * 16
                for k in range(16):
                    colv = jnp.full((16,), 0, jnp.int32) + (c0 + k)
                    x = plsc.load_gather(xb, [rowv, colv])
                    base, sig = _base_sig(x)
                    accs[k % 4] = accs[k % 4] + base * sig * sig
                return tuple(accs)

            zeros = jnp.zeros((16,), jnp.float32)
            a0, a1, a2, a3 = lax.fori_loop(0, N_COLS // 16, blk_body,
                                           (zeros, zeros, zeros, zeros))
            acc = (a0 + a1) + (a2 + a3)
            lbl = lb[pl.ds(g * 16, 16)]
            sco = sb[pl.ds(g * 16, 16)]
            wgt = wb[pl.ds(g * 16, 16)]
            mask = (lbl >= 0) & (lbl < N_COLS)
            safe = jnp.where(mask, lbl, 0)
            xp = plsc.load_gather(xb, [rowv, safe])
            bp, sp = _base_sig(xp)
            d = sco - sp
            corr = (bp - xp * sco) * d * d - bp * sp * sp
            tot = acc + jnp.where(mask, corr, jnp.float32(0))
            acc_ref[...] += tot * wgt

    issue(0, 0)

    def pair_body(i, carry):
        @pl.when(2 * i + 1 < nch)
        def _():
            issue(2 * i + 1, 1)
        wait(0)
        process(0)

        @pl.when(2 * i + 2 < nch)
        def _():
            issue(2 * i + 2, 0)

        @pl.when(2 * i + 1 < nch)
        def _():
            wait(1)
            process(1)
        return carry

    lax.fori_loop(0, MAX_CHUNKS_PER_WORKER // 2, pair_body, 0)

    pltpu.sync_copy(acc_ref, out_hbm.at[wid])


@functools.partial(jax.jit, static_argnames=())
def _qfl_partials(x, lbl, sco, wgt):
    kfn = pl.kernel(
        _qfl_body,
        out_type=jax.ShapeDtypeStruct((N_WORKERS, 16), jnp.float32),
        mesh=plsc.VectorSubcoreMesh(core_axis_name="c", subcore_axis_name="s"),
        compiler_params=pltpu.CompilerParams(needs_layout_passes=False),
        scratch_types=[
            pltpu.VMEM((CHUNK_ROWS, N_COLS), jnp.float32),
            pltpu.VMEM((CHUNK_ROWS, N_COLS), jnp.float32),
            pltpu.VMEM((CHUNK_ROWS,), jnp.int32),
            pltpu.VMEM((CHUNK_ROWS,), jnp.int32),
            pltpu.VMEM((CHUNK_ROWS,), jnp.float32),
            pltpu.VMEM((CHUNK_ROWS,), jnp.float32),
            pltpu.VMEM((CHUNK_ROWS,), jnp.float32),
            pltpu.VMEM((CHUNK_ROWS,), jnp.float32),
            pltpu.VMEM((16,), jnp.float32),
            pltpu.SemaphoreType.DMA,
            pltpu.SemaphoreType.DMA,
        ],
    )
    return kfn(x, lbl, sco, wgt)


def kernel(output, label, score, weight, avg_factor):
    partials = _qfl_partials(output, label.astype(jnp.int32), score, weight)
    return partials.sum() / avg_factor


# speedup vs baseline: 2.6332x; 1.6726x over previous
import functools

import jax
import jax.numpy as jnp
from jax import lax
from jax.experimental import pallas as pl
from jax.experimental.pallas import tpu as pltpu
from jax.experimental.pallas import tpu_sc as plsc

N_ROWS = 50000
N_COLS = 80
CHUNK_ROWS = 80
N_CHUNKS = N_ROWS // CHUNK_ROWS
N_WORKERS = 32
MAX_CHUNKS_PER_WORKER = 20

_L1P = (9.975032298825681e-06, 0.9992354838332771, -0.4902307234234269,
        0.28527268109062165, -0.13158182508881333, 0.03044900453868939)

_SIGN = -2147483648


def _poly_l1p(u):
    p = jnp.full((16,), _L1P[-1], jnp.float32)
    for c in _L1P[-2::-1]:
        p = p * u + jnp.float32(c)
    return p


def _base_sig(x):
    neg_abs = plsc.bitcast(plsc.bitcast(x, jnp.int32) | jnp.int32(_SIGN),
                           jnp.float32)
    u = jnp.exp(neg_abs)
    base = jnp.maximum(x, jnp.float32(0)) + _poly_l1p(u)
    r = jnp.float32(1) / (jnp.float32(1) + u)
    sig = r * jnp.exp(jnp.minimum(x, jnp.float32(0)))
    return base, sig


def _qfl_body(x_hbm, lbl_hbm, sco_hbm, wgt_hbm, out_hbm,
              xb0, xb1, lb0, lb1, sb0, sb1, wb0, wb1, acc_ref,
              sem0, sem1):
    core = lax.axis_index("c")
    sub = lax.axis_index("s")
    wid = sub * 2 + core
    nch = jnp.where(wid < 17, 20, 19)

    iota = lax.iota(jnp.int32, 16)
    acc_ref[...] = jnp.zeros((16,), jnp.float32)

    bufs = ((xb0, lb0, sb0, wb0, sem0), (xb1, lb1, sb1, wb1, sem1))

    def issue(n, slot):
        xb, lb, sb, wb, sem = bufs[slot]
        cid = wid + n * N_WORKERS
        r0 = cid * CHUNK_ROWS
        pltpu.async_copy(x_hbm.at[pl.ds(r0, CHUNK_ROWS)], xb, sem)
        pltpu.async_copy(lbl_hbm.at[pl.ds(r0, CHUNK_ROWS)], lb, sem)
        pltpu.async_copy(sco_hbm.at[pl.ds(r0, CHUNK_ROWS)], sb, sem)
        pltpu.async_copy(wgt_hbm.at[pl.ds(r0, CHUNK_ROWS)], wb, sem)

    def wait(slot):
        xb, lb, sb, wb, sem = bufs[slot]
        pltpu.make_async_copy(x_hbm.at[pl.ds(0, CHUNK_ROWS)], xb, sem).wait()
        pltpu.make_async_copy(lbl_hbm.at[pl.ds(0, CHUNK_ROWS)], lb, sem).wait()
        pltpu.make_async_copy(sco_hbm.at[pl.ds(0, CHUNK_ROWS)], sb, sem).wait()
        pltpu.make_async_copy(wgt_hbm.at[pl.ds(0, CHUNK_ROWS)], wb, sem).wait()

    def row_neg_sum(xb, j):
        terms = []
        for k in range(N_COLS // 16):
            x = xb[j, pl.ds(k * 16, 16)]
            base, sig = _base_sig(x)
            terms.append(base * sig * sig)
        return ((terms[0] + terms[1]) + (terms[2] + terms[3])) + terms[4]

    def process(slot):
        xb, lb, sb, wb, _ = bufs[slot]

        def row_body(i, carry):
            j = 2 * i
            t0 = row_neg_sum(xb, j)
            w0 = plsc.load_gather(wb, [jnp.full((16,), 0, jnp.int32) + j])
            t1 = row_neg_sum(xb, j + 1)
            w1 = plsc.load_gather(wb, [jnp.full((16,), 1, jnp.int32) + j])
            acc_ref[...] += t0 * w0 + t1 * w1
            return carry

        lax.fori_loop(0, CHUNK_ROWS // 2, row_body, 0)

        for g in range(CHUNK_ROWS // 16):
            rowv = g * 16 + iota
            lbl = lb[pl.ds(g * 16, 16)]
            sco = sb[pl.ds(g * 16, 16)]
            wgt = wb[pl.ds(g * 16, 16)]
            mask = (lbl >= 0) & (lbl < N_COLS)
            safe = jnp.where(mask, lbl, 0)
            xp = plsc.load_gather(xb, [rowv, safe])
            bp, sp = _base_sig(xp)
            d = sco - sp
            corr = (bp - xp * sco) * d * d - bp * sp * sp
            acc_ref[...] += jnp.where(mask, corr, jnp.float32(0)) * wgt

    issue(0, 0)

    def pair_body(i, carry):
        @pl.when(2 * i + 1 < nch)
        def _():
            issue(2 * i + 1, 1)
        wait(0)
        process(0)

        @pl.when(2 * i + 2 < nch)
        def _():
            issue(2 * i + 2, 0)

        @pl.when(2 * i + 1 < nch)
        def _():
            wait(1)
            process(1)
        return carry

    lax.fori_loop(0, MAX_CHUNKS_PER_WORKER // 2, pair_body, 0)

    pltpu.sync_copy(acc_ref, out_hbm.at[wid])


@functools.partial(jax.jit, static_argnames=())
def _qfl_partials(x, lbl, sco, wgt):
    kfn = pl.kernel(
        _qfl_body,
        out_type=jax.ShapeDtypeStruct((N_WORKERS, 16), jnp.float32),
        mesh=plsc.VectorSubcoreMesh(core_axis_name="c", subcore_axis_name="s"),
        compiler_params=pltpu.CompilerParams(needs_layout_passes=False),
        scratch_types=[
            pltpu.VMEM((CHUNK_ROWS, N_COLS), jnp.float32),
            pltpu.VMEM((CHUNK_ROWS, N_COLS), jnp.float32),
            pltpu.VMEM((CHUNK_ROWS,), jnp.int32),
            pltpu.VMEM((CHUNK_ROWS,), jnp.int32),
            pltpu.VMEM((CHUNK_ROWS,), jnp.float32),
            pltpu.VMEM((CHUNK_ROWS,), jnp.float32),
            pltpu.VMEM((CHUNK_ROWS,), jnp.float32),
            pltpu.VMEM((CHUNK_ROWS,), jnp.float32),
            pltpu.VMEM((16,), jnp.float32),
            pltpu.SemaphoreType.DMA,
            pltpu.SemaphoreType.DMA,
        ],
    )
    return kfn(x, lbl, sco, wgt)


def kernel(output, label, score, weight, avg_factor):
    partials = _qfl_partials(output, label.astype(jnp.int32), score, weight)
    return partials.sum() / avg_factor


# speedup vs baseline: 3.1028x; 1.1783x over previous
import functools

import jax
import jax.numpy as jnp
from jax import lax
from jax.experimental import pallas as pl
from jax.experimental.pallas import tpu as pltpu
from jax.experimental.pallas import tpu_sc as plsc

N_ROWS = 50000
N_COLS = 80
CHUNK_ROWS = 80
N_CHUNKS = N_ROWS // CHUNK_ROWS
N_WORKERS = 32
MAX_CHUNKS_PER_WORKER = 20

_L1P = (6.944574124645442e-05, 0.9962619482337944, -0.46644243862756857,
        0.21866548366222538, -0.055459313742082655)
_LOG2E = 1.4426950408889634

_SIGN = -2147483648


def _poly_l1p(u):
    p = jnp.full((16,), _L1P[-1], jnp.float32)
    for c in _L1P[-2::-1]:
        p = p * u + jnp.float32(c)
    return p


def _base_sig(x):
    neg_abs = plsc.bitcast(plsc.bitcast(x, jnp.int32) | jnp.int32(_SIGN),
                           jnp.float32)
    u = jnp.exp(neg_abs)
    base = jnp.maximum(x, jnp.float32(0)) + _poly_l1p(u)
    r = jnp.float32(1) / (jnp.float32(1) + u)
    sig = r * jnp.exp(jnp.minimum(x, jnp.float32(0)))
    return base, sig


def _qfl_body(x_hbm, lbl_hbm, sco_hbm, wgt_hbm, out_hbm,
              xb0, xb1, lb0, lb1, sb0, sb1, wb0, wb1, acc_ref,
              sem0, sem1):
    core = lax.axis_index("c")
    sub = lax.axis_index("s")
    wid = sub * 2 + core
    nch = jnp.where(wid < 17, 20, 19)

    iota = lax.iota(jnp.int32, 16)
    acc_ref[...] = jnp.zeros((16,), jnp.float32)

    bufs = ((xb0, lb0, sb0, wb0, sem0), (xb1, lb1, sb1, wb1, sem1))

    def issue(n, slot):
        xb, lb, sb, wb, sem = bufs[slot]
        cid = wid + n * N_WORKERS
        r0 = cid * CHUNK_ROWS
        pltpu.async_copy(x_hbm.at[pl.ds(r0, CHUNK_ROWS)], xb, sem)
        pltpu.async_copy(lbl_hbm.at[pl.ds(r0, CHUNK_ROWS)], lb, sem)
        pltpu.async_copy(sco_hbm.at[pl.ds(r0, CHUNK_ROWS)], sb, sem)
        pltpu.async_copy(wgt_hbm.at[pl.ds(r0, CHUNK_ROWS)], wb, sem)

    def wait(slot):
        xb, lb, sb, wb, sem = bufs[slot]
        pltpu.make_async_copy(x_hbm.at[pl.ds(0, CHUNK_ROWS)], xb, sem).wait()
        pltpu.make_async_copy(lbl_hbm.at[pl.ds(0, CHUNK_ROWS)], lb, sem).wait()
        pltpu.make_async_copy(sco_hbm.at[pl.ds(0, CHUNK_ROWS)], sb, sem).wait()
        pltpu.make_async_copy(wgt_hbm.at[pl.ds(0, CHUNK_ROWS)], wb, sem).wait()

    def row_neg_sum(xb, j):
        terms = []
        for k in range(N_COLS // 16):
            x = xb[j, pl.ds(k * 16, 16)]
            base, sig = _base_sig(x)
            terms.append(base * sig * sig)
        return ((terms[0] + terms[1]) + (terms[2] + terms[3])) + terms[4]

    def process(slot):
        xb, lb, sb, wb, _ = bufs[slot]

        def row_body(i, acc):
            j = 2 * i
            t0 = row_neg_sum(xb, j)
            w0 = plsc.load_gather(wb, [jnp.full((16,), 0, jnp.int32) + j])
            t1 = row_neg_sum(xb, j + 1)
            w1 = plsc.load_gather(wb, [jnp.full((16,), 1, jnp.int32) + j])
            return acc + (t0 * w0 + t1 * w1)

        acc = lax.fori_loop(0, CHUNK_ROWS // 2, row_body,
                            jnp.zeros((16,), jnp.float32))

        for g in range(CHUNK_ROWS // 16):
            rowv = g * 16 + iota
            lbl = lb[pl.ds(g * 16, 16)]
            sco = sb[pl.ds(g * 16, 16)]
            wgt = wb[pl.ds(g * 16, 16)]
            mask = (lbl >= 0) & (lbl < N_COLS)
            safe = jnp.where(mask, lbl, 0)
            xp = plsc.load_gather(xb, [rowv, safe])
            bp, sp = _base_sig(xp)
            d = sco - sp
            corr = (bp - xp * sco) * d * d - bp * sp * sp
            acc = acc + jnp.where(mask, corr, jnp.float32(0)) * wgt
        acc_ref[...] += acc

    issue(0, 0)

    def pair_body(i, carry):
        @pl.when(2 * i + 1 < nch)
        def _():
            issue(2 * i + 1, 1)
        wait(0)
        process(0)

        @pl.when(2 * i + 2 < nch)
        def _():
            issue(2 * i + 2, 0)

        @pl.when(2 * i + 1 < nch)
        def _():
            wait(1)
            process(1)
        return carry

    lax.fori_loop(0, MAX_CHUNKS_PER_WORKER // 2, pair_body, 0)

    pltpu.sync_copy(acc_ref, out_hbm.at[wid])


@functools.partial(jax.jit, static_argnames=())
def _qfl_partials(x, lbl, sco, wgt):
    kfn = pl.kernel(
        _qfl_body,
        out_type=jax.ShapeDtypeStruct((N_WORKERS, 16), jnp.float32),
        mesh=plsc.VectorSubcoreMesh(core_axis_name="c", subcore_axis_name="s"),
        compiler_params=pltpu.CompilerParams(needs_layout_passes=False),
        scratch_types=[
            pltpu.VMEM((CHUNK_ROWS, N_COLS), jnp.float32),
            pltpu.VMEM((CHUNK_ROWS, N_COLS), jnp.float32),
            pltpu.VMEM((CHUNK_ROWS,), jnp.int32),
            pltpu.VMEM((CHUNK_ROWS,), jnp.int32),
            pltpu.VMEM((CHUNK_ROWS,), jnp.float32),
            pltpu.VMEM((CHUNK_ROWS,), jnp.float32),
            pltpu.VMEM((CHUNK_ROWS,), jnp.float32),
            pltpu.VMEM((CHUNK_ROWS,), jnp.float32),
            pltpu.VMEM((16,), jnp.float32),
            pltpu.SemaphoreType.DMA,
            pltpu.SemaphoreType.DMA,
        ],
    )
    return kfn(x, lbl, sco, wgt)


def kernel(output, label, score, weight, avg_factor):
    partials = _qfl_partials(output, label.astype(jnp.int32), score, weight)
    return partials.sum() / avg_factor


# speedup vs baseline: 3.1055x; 1.0009x over previous
import functools

import jax
import jax.numpy as jnp
from jax import lax
from jax.experimental import pallas as pl
from jax.experimental.pallas import tpu as pltpu
from jax.experimental.pallas import tpu_sc as plsc

N_ROWS = 50000
N_COLS = 80
CHUNK_ROWS = 80
N_CHUNKS = N_ROWS // CHUNK_ROWS
N_WORKERS = 32
MAX_CHUNKS_PER_WORKER = 20

_L1P = (6.944574124645442e-05, 0.9962619482337944, -0.46644243862756857,
        0.21866548366222538, -0.055459313742082655)
_LOG2E = 1.4426950408889634

_SIGN = -2147483648


def _poly_l1p(u):
    p = jnp.full((16,), _L1P[-1], jnp.float32)
    for c in _L1P[-2::-1]:
        p = p * u + jnp.float32(c)
    return p


def _base_sig(x):
    neg_abs = plsc.bitcast(plsc.bitcast(x, jnp.int32) | jnp.int32(_SIGN),
                           jnp.float32)
    u = jnp.exp(neg_abs)
    base = jnp.maximum(x, jnp.float32(0)) + _poly_l1p(u)
    r = jnp.float32(1) / (jnp.float32(1) + u)
    sig = r * jnp.exp(jnp.minimum(x, jnp.float32(0)))
    return base, sig


def _qfl_body(x_hbm, lbl_hbm, sco_hbm, wgt_hbm, out_hbm,
              xb0, xb1, lb0, lb1, sb0, sb1, wb0, wb1, acc_ref,
              sem0, sem1):
    core = lax.axis_index("c")
    sub = lax.axis_index("s")
    wid = sub * 2 + core
    nch = jnp.where(wid < 17, 20, 19)

    iota = lax.iota(jnp.int32, 16)
    acc_ref[...] = jnp.zeros((16,), jnp.float32)

    bufs = ((xb0, lb0, sb0, wb0, sem0), (xb1, lb1, sb1, wb1, sem1))

    def issue(n, slot):
        xb, lb, sb, wb, sem = bufs[slot]
        cid = wid + n * N_WORKERS
        r0 = cid * CHUNK_ROWS
        pltpu.async_copy(x_hbm.at[pl.ds(r0, CHUNK_ROWS)], xb, sem)
        pltpu.async_copy(lbl_hbm.at[pl.ds(r0, CHUNK_ROWS)], lb, sem)
        pltpu.async_copy(sco_hbm.at[pl.ds(r0, CHUNK_ROWS)], sb, sem)
        pltpu.async_copy(wgt_hbm.at[pl.ds(r0, CHUNK_ROWS)], wb, sem)

    def wait(slot):
        xb, lb, sb, wb, sem = bufs[slot]
        pltpu.make_async_copy(x_hbm.at[pl.ds(0, CHUNK_ROWS)], xb, sem).wait()
        pltpu.make_async_copy(lbl_hbm.at[pl.ds(0, CHUNK_ROWS)], lb, sem).wait()
        pltpu.make_async_copy(sco_hbm.at[pl.ds(0, CHUNK_ROWS)], sb, sem).wait()
        pltpu.make_async_copy(wgt_hbm.at[pl.ds(0, CHUNK_ROWS)], wb, sem).wait()

    def row_neg_sum(xb, j):
        terms = []
        for k in range(N_COLS // 16):
            x = xb[j, pl.ds(k * 16, 16)]
            base, sig = _base_sig(x)
            terms.append(base * sig * sig)
        return ((terms[0] + terms[1]) + (terms[2] + terms[3])) + terms[4]

    def process(slot):
        xb, lb, sb, wb, _ = bufs[slot]

        def row_body(i, acc):
            j = 2 * i
            t0 = row_neg_sum(xb, j)
            w0 = plsc.load_gather(wb, [jnp.full((16,), 0, jnp.int32) + j])
            t1 = row_neg_sum(xb, j + 1)
            w1 = plsc.load_gather(wb, [jnp.full((16,), 1, jnp.int32) + j])
            return acc + (t0 * w0 + t1 * w1)

        acc = lax.fori_loop(0, CHUNK_ROWS // 2, row_body,
                            jnp.zeros((16,), jnp.float32))

        for g in range(CHUNK_ROWS // 16):
            rowv = g * 16 + iota
            lbl = lb[pl.ds(g * 16, 16)]
            sco = sb[pl.ds(g * 16, 16)]
            wgt = wb[pl.ds(g * 16, 16)]
            mask = (lbl >= 0) & (lbl < N_COLS)
            safe = jnp.where(mask, lbl, 0)
            xp = plsc.load_gather(xb, [rowv, safe])
            bp, sp = _base_sig(xp)
            d = sco - sp
            corr = (bp - xp * sco) * d * d - bp * sp * sp
            acc = acc + jnp.where(mask, corr, jnp.float32(0)) * wgt
        acc_ref[...] += acc

    issue(0, 0)

    def pair_body(i, carry):
        @pl.when(2 * i + 1 < nch)
        def _():
            issue(2 * i + 1, 1)
        wait(0)
        process(0)

        @pl.when(2 * i + 2 < nch)
        def _():
            issue(2 * i + 2, 0)

        @pl.when(2 * i + 1 < nch)
        def _():
            wait(1)
            process(1)
        return carry

    lax.fori_loop(0, MAX_CHUNKS_PER_WORKER // 2, pair_body, 0)

    pltpu.sync_copy(acc_ref, out_hbm.at[wid])


@functools.partial(jax.jit, static_argnames=())
def _qfl_partials(x, lbl, sco, wgt):
    kfn = pl.kernel(
        _qfl_body,
        out_type=jax.ShapeDtypeStruct((N_WORKERS, 16), jnp.float32),
        mesh=plsc.VectorSubcoreMesh(core_axis_name="c", subcore_axis_name="s"),
        compiler_params=pltpu.CompilerParams(needs_layout_passes=False,
                                             use_tc_tiling_on_sc=True),
        scratch_types=[
            pltpu.VMEM((CHUNK_ROWS, N_COLS), jnp.float32),
            pltpu.VMEM((CHUNK_ROWS, N_COLS), jnp.float32),
            pltpu.VMEM((CHUNK_ROWS,), jnp.int32),
            pltpu.VMEM((CHUNK_ROWS,), jnp.int32),
            pltpu.VMEM((CHUNK_ROWS,), jnp.float32),
            pltpu.VMEM((CHUNK_ROWS,), jnp.float32),
            pltpu.VMEM((CHUNK_ROWS,), jnp.float32),
            pltpu.VMEM((CHUNK_ROWS,), jnp.float32),
            pltpu.VMEM((16,), jnp.float32),
            pltpu.SemaphoreType.DMA,
            pltpu.SemaphoreType.DMA,
        ],
    )
    return kfn(x, lbl, sco, wgt)


def kernel(output, label, score, weight, avg_factor):
    partials = _qfl_partials(output, label.astype(jnp.int32), score, weight)
    return partials.sum() / avg_factor
